# baseline (speedup 1.0000x reference)
import functools

import jax
import jax.numpy as jnp
from jax.experimental import pallas as pl
from jax.experimental.pallas import tpu as pltpu

_N_CHUNKS = 5


def _gcn_block(adj_ref, x_hbm, w_ref, out_ref,
               wb_ref, st_ref, support_ref, sem0, sem1):
    k_rows = x_hbm.shape[0]
    ch = k_rows // _N_CHUNKS
    sems = (sem0, sem1)

    @pl.when(pl.program_id(0) == 0)
    def _():
        wb_ref[...] = w_ref[...].astype(jnp.bfloat16)
        pltpu.make_async_copy(
            x_hbm.at[pl.ds(0, ch), :], st_ref.at[0], sems[0]).start()
        for c in range(_N_CHUNKS):
            cur = c % 2
            if c + 1 < _N_CHUNKS:
                nxt = (c + 1) % 2
                pltpu.make_async_copy(
                    x_hbm.at[pl.ds((c + 1) * ch, ch), :],
                    st_ref.at[nxt], sems[nxt]).start()
            pltpu.make_async_copy(
                x_hbm.at[pl.ds(c * ch, ch), :], st_ref.at[cur],
                sems[cur]).wait()
            support_ref[pl.ds(c * ch, ch), :] = jnp.dot(
                st_ref[cur].astype(jnp.bfloat16),
                wb_ref[...],
                preferred_element_type=jnp.float32,
            ).astype(jnp.bfloat16)

    out_ref[...] = jnp.dot(
        adj_ref[...].astype(jnp.bfloat16),
        support_ref[...],
        preferred_element_type=jnp.float32,
    )


@functools.partial(jax.jit, static_argnames=("block_m",))
def _gcn(adj, x, W, block_m=200):
    m, k = adj.shape
    d_in, d_out = W.shape
    bm = min(block_m, m)
    return pl.pallas_call(
        _gcn_block,
        grid=(pl.cdiv(m, bm),),
        in_specs=[
            pl.BlockSpec((bm, k), lambda i: (i, 0)),
            pl.BlockSpec(memory_space=pl.ANY),
            pl.BlockSpec((d_in, d_out), lambda i: (0, 0)),
        ],
        out_specs=pl.BlockSpec((bm, d_out), lambda i: (i, 0)),
        out_shape=jax.ShapeDtypeStruct((m, d_out), jnp.float32),
        scratch_shapes=[
            pltpu.VMEM((d_in, d_out), jnp.bfloat16),
            pltpu.VMEM((2, x.shape[0] // _N_CHUNKS, d_in), jnp.float32),
            pltpu.VMEM((x.shape[0], d_out), jnp.bfloat16),
            pltpu.SemaphoreType.DMA,
            pltpu.SemaphoreType.DMA,
        ],
    )(adj, x, W)


def kernel(adj, x, W):
    return _gcn(adj, x, W)


# speedup vs baseline: 1.0689x; 1.0689x over previous
import functools

import jax
import jax.numpy as jnp
from jax.experimental import pallas as pl
from jax.experimental.pallas import tpu as pltpu

_N_CHUNKS = 10


def _gcn_block(adj_ref, x_hbm, w_ref, out_ref,
               wb_ref, st_ref, support_ref, sem0, sem1):
    k_rows = x_hbm.shape[0]
    ch = k_rows // _N_CHUNKS
    sems = (sem0, sem1)

    @pl.when(pl.program_id(0) == 0)
    def _():
        wb_ref[...] = w_ref[...].astype(jnp.bfloat16)
        pltpu.make_async_copy(
            x_hbm.at[pl.ds(0, ch), :], st_ref.at[0], sems[0]).start()
        for c in range(_N_CHUNKS):
            cur = c % 2
            if c + 1 < _N_CHUNKS:
                nxt = (c + 1) % 2
                pltpu.make_async_copy(
                    x_hbm.at[pl.ds((c + 1) * ch, ch), :],
                    st_ref.at[nxt], sems[nxt]).start()
            pltpu.make_async_copy(
                x_hbm.at[pl.ds(c * ch, ch), :], st_ref.at[cur],
                sems[cur]).wait()
            support_ref[pl.ds(c * ch, ch), :] = jnp.dot(
                st_ref[cur].astype(jnp.bfloat16),
                wb_ref[...],
                preferred_element_type=jnp.float32,
            ).astype(jnp.bfloat16)

    out_ref[...] = jnp.dot(
        adj_ref[...].astype(jnp.bfloat16),
        support_ref[...],
        preferred_element_type=jnp.float32,
    )


@functools.partial(jax.jit, static_argnames=("block_m",))
def _gcn(adj, x, W, block_m=400):
    m, k = adj.shape
    d_in, d_out = W.shape
    bm = min(block_m, m)
    return pl.pallas_call(
        _gcn_block,
        grid=(pl.cdiv(m, bm),),
        in_specs=[
            pl.BlockSpec((bm, k), lambda i: (i, 0)),
            pl.BlockSpec(memory_space=pl.ANY),
            pl.BlockSpec((d_in, d_out), lambda i: (0, 0)),
        ],
        out_specs=pl.BlockSpec((bm, d_out), lambda i: (i, 0)),
        out_shape=jax.ShapeDtypeStruct((m, d_out), jnp.float32),
        scratch_shapes=[
            pltpu.VMEM((d_in, d_out), jnp.bfloat16),
            pltpu.VMEM((2, x.shape[0] // _N_CHUNKS, d_in), jnp.float32),
            pltpu.VMEM((x.shape[0], d_out), jnp.bfloat16),
            pltpu.SemaphoreType.DMA,
            pltpu.SemaphoreType.DMA,
        ],
    )(adj, x, W)


def kernel(adj, x, W):
    return _gcn(adj, x, W)
